# Initial kernel scaffold; baseline (speedup 1.0000x reference)
import jax
import jax.numpy as jnp
from jax.experimental import pallas as pl


def kernel(ids, table):
    raise NotImplementedError("write your pallas kernel here")


# speedup vs baseline: 1.4492x; 1.4492x over previous
import functools

import jax
import jax.numpy as jnp
from jax import lax
from jax.experimental import pallas as pl
from jax.experimental.pallas import tpu as pltpu
from jax.experimental.pallas import tpu_sc as plsc

_NC = 2
_NS = 16
_NW = _NC * _NS
_IDXW = 128
_GROUP_IDX_ROWS = 4
_GROUP_ROWS = _GROUP_IDX_ROWS * _IDXW


def _make_embed(n_idx_rows_per_w, n_groups, dim, vocab):
    mesh = plsc.VectorSubcoreMesh(core_axis_name="c", subcore_axis_name="s")
    b_per_w = n_idx_rows_per_w * _IDXW
    total_rows = b_per_w * _NW

    @functools.partial(
        pl.kernel,
        out_type=jax.ShapeDtypeStruct((total_rows, dim), jnp.float32),
        mesh=mesh,
        compiler_params=pltpu.CompilerParams(use_tc_tiling_on_sc=False),
        scratch_types=[
            pltpu.VMEM((n_idx_rows_per_w, _IDXW), jnp.int32),
            pltpu.VMEM((_GROUP_ROWS, dim), jnp.float32),
            pltpu.SemaphoreType.DMA,
        ],
    )
    def embed(table_hbm, ids_hbm, out_hbm, idx_v, rows_v, sem):
        wid = lax.axis_index("s") * _NC + lax.axis_index("c")
        pltpu.sync_copy(
            ids_hbm.at[pl.ds(wid * n_idx_rows_per_w, n_idx_rows_per_w)], idx_v
        )
        out_base = wid * b_per_w

        def grp(g, carry):
            handles = []
            for j in range(_GROUP_IDX_ROWS):
                h = pltpu.async_copy(
                    table_hbm.at[idx_v.at[g * _GROUP_IDX_ROWS + j]],
                    rows_v.at[pl.ds(j * _IDXW, _IDXW)],
                    sem,
                )
                handles.append(h)
            for h in handles:
                h.wait()
            pltpu.sync_copy(
                rows_v, out_hbm.at[pl.ds(out_base + g * _GROUP_ROWS, _GROUP_ROWS)]
            )
            return carry

        lax.fori_loop(0, n_groups, grp, 0)

    return embed


def kernel(ids, table):
    orig_shape = ids.shape
    dim = table.shape[1]
    b = ids.size
    assert b % (_NW * _IDXW) == 0
    n_idx_rows_per_w = b // (_NW * _IDXW)
    assert n_idx_rows_per_w % _GROUP_IDX_ROWS == 0
    n_groups = n_idx_rows_per_w // _GROUP_IDX_ROWS
    ids_flat = ids.reshape(_NW * n_idx_rows_per_w, _IDXW)
    embed = _make_embed(n_idx_rows_per_w, n_groups, dim, table.shape[0])
    out = embed(table, ids_flat)
    return out.reshape(*orig_shape, dim)


# speedup vs baseline: 1.5019x; 1.0363x over previous
import functools

import jax
import jax.numpy as jnp
from jax import lax
from jax.experimental import pallas as pl
from jax.experimental.pallas import tpu as pltpu
from jax.experimental.pallas import tpu_sc as plsc

_NC = 2
_NS = 16
_NW = _NC * _NS
_IDXW = 128
_NBUF = 8


def _make_embed(n_idx_rows_per_w, dim):
    mesh = plsc.VectorSubcoreMesh(core_axis_name="c", subcore_axis_name="s")
    b_per_w = n_idx_rows_per_w * _IDXW
    total_rows = b_per_w * _NW

    @functools.partial(
        pl.kernel,
        out_type=jax.ShapeDtypeStruct((total_rows, dim), jnp.float32),
        mesh=mesh,
        compiler_params=pltpu.CompilerParams(use_tc_tiling_on_sc=False),
        scratch_types=[
            pltpu.VMEM((n_idx_rows_per_w, _IDXW), jnp.int32),
            pltpu.VMEM((_NBUF, _IDXW, dim), jnp.float32),
        ]
        + [pltpu.SemaphoreType.DMA] * _NBUF,
    )
    def embed(table_hbm, ids_hbm, out_hbm, idx_v, rows_v, *sems):
        wid = lax.axis_index("s") * _NC + lax.axis_index("c")
        pltpu.sync_copy(
            ids_hbm.at[pl.ds(wid * n_idx_rows_per_w, n_idx_rows_per_w)], idx_v
        )
        out_base = wid * b_per_w

        def fire(row, b):
            pltpu.async_copy(table_hbm.at[idx_v.at[row]], rows_v.at[b], sems[b])

        def drain(b):
            pltpu.make_async_copy(
                table_hbm.at[pl.ds(0, _IDXW)], rows_v.at[b], sems[b]
            ).wait()

        for b in range(_NBUF):
            fire(b, b)

        def step(it, carry):
            gg = it * _NBUF
            for b in range(_NBUF):
                g = gg + b
                drain(b)
                pltpu.sync_copy(
                    rows_v.at[b], out_hbm.at[pl.ds(out_base + g * _IDXW, _IDXW)]
                )
                fire(g + _NBUF, b)
            return carry

        n_full = n_idx_rows_per_w // _NBUF - 1
        lax.fori_loop(0, n_full, step, 0)

        tail = (n_idx_rows_per_w // _NBUF - 1) * _NBUF
        for b in range(_NBUF):
            g = tail + b
            drain(b)
            pltpu.sync_copy(
                rows_v.at[b], out_hbm.at[pl.ds(out_base + g * _IDXW, _IDXW)]
            )

    return embed


def kernel(ids, table):
    orig_shape = ids.shape
    dim = table.shape[1]
    b = ids.size
    assert b % (_NW * _IDXW) == 0
    n_idx_rows_per_w = b // (_NW * _IDXW)
    assert n_idx_rows_per_w % _NBUF == 0
    ids_flat = ids.reshape(_NW * n_idx_rows_per_w, _IDXW)
    embed = _make_embed(n_idx_rows_per_w, dim)
    out = embed(table, ids_flat)
    return out.reshape(*orig_shape, dim)


# speedup vs baseline: 2.0508x; 1.3655x over previous
import functools

import jax
import jax.numpy as jnp
from jax import lax
from jax.experimental import pallas as pl
from jax.experimental.pallas import tpu as pltpu
from jax.experimental.pallas import tpu_sc as plsc

_NC = 2
_NS = 16
_NW = _NC * _NS
_IDXW = 128
_NBUF = 8


def _make_embed(n_idx_rows_per_w, dim):
    mesh = plsc.VectorSubcoreMesh(core_axis_name="c", subcore_axis_name="s")
    b_per_w = n_idx_rows_per_w * _IDXW
    total_rows = b_per_w * _NW

    @functools.partial(
        pl.kernel,
        out_type=jax.ShapeDtypeStruct((total_rows, 128), jnp.float32),
        mesh=mesh,
        compiler_params=pltpu.CompilerParams(use_tc_tiling_on_sc=False),
        scratch_types=[
            pltpu.VMEM((n_idx_rows_per_w, _IDXW), jnp.int32),
            pltpu.VMEM((_NBUF, _IDXW, dim), jnp.float32),
        ]
        + [pltpu.SemaphoreType.DMA] * _NBUF,
    )
    def embed(table_hbm, ids_hbm, out_hbm, idx_v, rows_v, *sems):
        wid = lax.axis_index("s") * _NC + lax.axis_index("c")
        pltpu.sync_copy(
            ids_hbm.at[pl.ds(wid * n_idx_rows_per_w, n_idx_rows_per_w)], idx_v
        )
        out_base = wid * b_per_w

        def fire(row, b):
            pltpu.async_copy(table_hbm.at[idx_v.at[row]], rows_v.at[b], sems[b])

        def drain(b):
            pltpu.make_async_copy(
                table_hbm.at[pl.ds(0, _IDXW)], rows_v.at[b], sems[b]
            ).wait()

        for b in range(_NBUF):
            fire(b, b)

        def step(it, carry):
            gg = it * _NBUF
            for b in range(_NBUF):
                g = gg + b
                drain(b)
                pltpu.sync_copy(
                    rows_v.at[b],
                    out_hbm.at[pl.ds(out_base + g * _IDXW, _IDXW)].at[:, pl.ds(0, dim)],
                )
                fire(g + _NBUF, b)
            return carry

        n_full = n_idx_rows_per_w // _NBUF - 1
        lax.fori_loop(0, n_full, step, 0)

        tail = (n_idx_rows_per_w // _NBUF - 1) * _NBUF
        for b in range(_NBUF):
            g = tail + b
            drain(b)
            pltpu.sync_copy(
                rows_v.at[b],
                out_hbm.at[pl.ds(out_base + g * _IDXW, _IDXW)].at[:, pl.ds(0, dim)],
            )

    return embed


def kernel(ids, table):
    orig_shape = ids.shape
    dim = table.shape[1]
    b = ids.size
    assert b % (_NW * _IDXW) == 0
    n_idx_rows_per_w = b // (_NW * _IDXW)
    assert n_idx_rows_per_w % _NBUF == 0
    ids_flat = ids.reshape(_NW * n_idx_rows_per_w, _IDXW)
    embed = _make_embed(n_idx_rows_per_w, dim)
    out = embed(table, ids_flat)
    return out[:, :dim].reshape(*orig_shape, dim)
